# baseline (speedup 1.0000x reference)
import functools
import math

import jax
import jax.numpy as jnp
from jax import lax
from jax.experimental import pallas as pl
from jax.experimental.pallas import tpu as pltpu
from jax.experimental.pallas import tpu_sc as plsc

D_MODEL = 64
SCALE = math.sqrt(D_MODEL)
LANES = 16
PADW = 128

NUM_CORES = 2
NUM_SUBCORES = 16
NW = NUM_CORES * NUM_SUBCORES

BBLK = 128


@functools.partial(jax.jit, static_argnums=(2, 3))
def _embed(xT, lut_p, S, B):
    mesh = plsc.VectorSubcoreMesh(core_axis_name="c", subcore_axis_name="s")

    @functools.partial(
        pl.kernel,
        out_type=jax.ShapeDtypeStruct((S, D_MODEL, B), jnp.float32),
        mesh=mesh,
        scratch_types=[
            pltpu.VMEM((S, BBLK), jnp.int32),
            pltpu.VMEM((BBLK, PADW), jnp.float32),
            pltpu.VMEM((BBLK, PADW), jnp.float32),
            pltpu.VMEM((D_MODEL, BBLK), jnp.float32),
            pltpu.VMEM((D_MODEL, BBLK), jnp.float32),
            pltpu.SemaphoreType.DMA,
            pltpu.SemaphoreType.DMA,
            pltpu.SemaphoreType.DMA,
            pltpu.SemaphoreType.DMA,
        ],
        compiler_params=pltpu.CompilerParams(
            use_tc_tiling_on_sc=True, needs_layout_passes=False
        ),
    )
    def emb_kernel(x_hbm, lut_hbm, out_hbm, idx_all, rows_a, rows_b,
                   out_a, out_b, gs_a, gs_b, os_a, os_b):
        w = lax.axis_index("s") * NUM_CORES + lax.axis_index("c")
        b0 = w * BBLK
        pltpu.sync_copy(x_hbm.at[:, pl.ds(b0, BBLK)], idx_all)

        iot = lax.iota(jnp.int32, LANES)

        def gstart(s, rows_ref, sem):
            pltpu.async_copy(lut_hbm.at[idx_all.at[s]], rows_ref, sem)

        def gwait(s, rows_ref, sem):
            pltpu.make_async_copy(lut_hbm.at[idx_all.at[s]], rows_ref, sem).wait()

        def ostart(s, out_ref, sem):
            pltpu.async_copy(out_ref, out_hbm.at[s, :, pl.ds(b0, BBLK)], sem)

        def owait(s, out_ref, sem):
            pltpu.make_async_copy(out_ref, out_hbm.at[s, :, pl.ds(b0, BBLK)], sem).wait()

        def transpose_scale(rows_ref, out_ref):
            def dbody(d, c):
                dcol = jnp.zeros((LANES,), jnp.int32) + d
                for jg in range(BBLK // LANES):
                    v = plsc.load_gather(rows_ref, [iot + (jg * LANES), dcol])
                    out_ref[d, pl.ds(jg * LANES, LANES)] = v * SCALE
                return c
            lax.fori_loop(0, D_MODEL, dbody, 0)

        gstart(0, rows_a, gs_a)
        gstart(1, rows_b, gs_b)

        def kbody(k, c):
            s = 2 * k
            gwait(s, rows_a, gs_a)

            @pl.when(k > 0)
            def _():
                owait(s - 2, out_a, os_a)

            transpose_scale(rows_a, out_a)
            ostart(s, out_a, os_a)

            @pl.when(k < (S // 2) - 1)
            def _():
                gstart(s + 2, rows_a, gs_a)

            gwait(s + 1, rows_b, gs_b)

            @pl.when(k > 0)
            def _():
                owait(s - 1, out_b, os_b)

            transpose_scale(rows_b, out_b)
            ostart(s + 1, out_b, os_b)

            @pl.when(k < (S // 2) - 1)
            def _():
                gstart(s + 3, rows_b, gs_b)

            return c

        lax.fori_loop(0, S // 2, kbody, 0)
        owait(S - 2, out_a, os_a)
        owait(S - 1, out_b, os_b)

    return emb_kernel(xT, lut_p)


def kernel(x, lut):
    b, s = x.shape
    xT = jnp.swapaxes(x, 0, 1).astype(jnp.int32)
    lut_p = jnp.pad(lut, ((0, 0), (0, PADW - D_MODEL)))
    out = _embed(xT, lut_p, s, b)
    return jnp.transpose(out, (2, 0, 1))


# speedup vs baseline: 1.4979x; 1.4979x over previous
import functools
import math

import jax
import jax.numpy as jnp
from jax import lax
from jax.experimental import pallas as pl
from jax.experimental.pallas import tpu as pltpu
from jax.experimental.pallas import tpu_sc as plsc

D_MODEL = 64
SCALE = math.sqrt(D_MODEL)
LANES = 16
PADW = 128

NUM_CORES = 2
NUM_SUBCORES = 16
NW = NUM_CORES * NUM_SUBCORES

BBLK = 128


@functools.partial(jax.jit, static_argnums=(2, 3))
def _embed(xT, lut2, S, B):
    mesh = plsc.VectorSubcoreMesh(core_axis_name="c", subcore_axis_name="s")

    @functools.partial(
        pl.kernel,
        out_type=jax.ShapeDtypeStruct((S, D_MODEL, B), jnp.float32),
        mesh=mesh,
        scratch_types=[
            pltpu.VMEM((S, BBLK), jnp.int32),
            pltpu.VMEM((BBLK,), jnp.int32),
            pltpu.VMEM((BBLK,), jnp.int32),
            pltpu.VMEM((BBLK, PADW), jnp.float32),
            pltpu.VMEM((BBLK, PADW), jnp.float32),
            pltpu.VMEM((D_MODEL, BBLK), jnp.float32),
            pltpu.VMEM((D_MODEL, BBLK), jnp.float32),
            pltpu.SemaphoreType.DMA,
            pltpu.SemaphoreType.DMA,
            pltpu.SemaphoreType.DMA,
            pltpu.SemaphoreType.DMA,
        ],
        compiler_params=pltpu.CompilerParams(
            use_tc_tiling_on_sc=True, needs_layout_passes=False
        ),
    )
    def emb_kernel(x_hbm, lut_hbm, out_hbm, idx_all, pair_a, pair_b,
                   rows_a, rows_b, out_a, out_b, gs_a, gs_b, os_a, os_b):
        w = lax.axis_index("s") * NUM_CORES + lax.axis_index("c")
        b0 = w * BBLK
        pltpu.sync_copy(x_hbm.at[:, pl.ds(b0, BBLK)], idx_all)

        iot = lax.iota(jnp.int32, LANES)
        rots = [jnp.bitwise_and(iot + k, LANES - 1) for k in range(LANES)]

        def gstart(s, pair_ref, rows_ref, sem):
            for jg in range(BBLK // LANES):
                sl = pl.ds(jg * LANES, LANES)
                pair_ref[sl] = lax.shift_right_logical(idx_all[s, sl], 1)
            pltpu.async_copy(lut_hbm.at[pair_ref], rows_ref, sem)

        def gwait(pair_ref, rows_ref, sem):
            pltpu.make_async_copy(lut_hbm.at[pair_ref], rows_ref, sem).wait()

        def ostart(s, out_ref, sem):
            pltpu.async_copy(out_ref, out_hbm.at[s, :, pl.ds(b0, BBLK)], sem)

        def owait(s, out_ref, sem):
            pltpu.make_async_copy(out_ref, out_hbm.at[s, :, pl.ds(b0, BBLK)], sem).wait()

        def transpose_scale(s, rows_ref, out_ref):
            halfs = []
            rowvs = []
            for jg in range(BBLK // LANES):
                sl = pl.ds(jg * LANES, LANES)
                halfs.append(lax.shift_left(jnp.bitwise_and(idx_all[s, sl], 1), 6))
                rowvs.append(iot + (jg * LANES))

            def dblock(db, c):
                d0 = db * LANES
                for jg in range(BBLK // LANES):
                    base = halfs[jg] + d0
                    for k in range(LANES):
                        v = plsc.load_gather(rows_ref, [rowvs[jg], base + rots[k]])
                        plsc.store_scatter(out_ref, [rots[k] + d0, rowvs[jg]], v * SCALE)
                return c

            lax.fori_loop(0, D_MODEL // LANES, dblock, 0)

        gstart(0, pair_a, rows_a, gs_a)
        gstart(1, pair_b, rows_b, gs_b)

        def kbody(k, c):
            s = 2 * k
            gwait(pair_a, rows_a, gs_a)

            @pl.when(k > 0)
            def _():
                owait(s - 2, out_a, os_a)

            transpose_scale(s, rows_a, out_a)
            ostart(s, out_a, os_a)

            @pl.when(k < (S // 2) - 1)
            def _():
                gstart(s + 2, pair_a, rows_a, gs_a)

            gwait(pair_b, rows_b, gs_b)

            @pl.when(k > 0)
            def _():
                owait(s - 1, out_b, os_b)

            transpose_scale(s + 1, rows_b, out_b)
            ostart(s + 1, out_b, os_b)

            @pl.when(k < (S // 2) - 1)
            def _():
                gstart(s + 3, pair_b, rows_b, gs_b)

            return c

        lax.fori_loop(0, S // 2, kbody, 0)
        owait(S - 2, out_a, os_a)
        owait(S - 1, out_b, os_b)

    return emb_kernel(xT, lut2)


def kernel(x, lut):
    b, s = x.shape
    v, d = lut.shape
    xT = jnp.swapaxes(x, 0, 1).astype(jnp.int32)
    lut2 = jnp.reshape(lut, (v // 2, 2 * d))
    out = _embed(xT, lut2, s, b)
    return jnp.transpose(out, (2, 0, 1))
